# baseline (speedup 1.0000x reference)
import functools

import jax
import jax.numpy as jnp
from jax import lax
from jax.experimental import pallas as pl
from jax.experimental.pallas import tpu as pltpu
from jax.experimental.pallas import tpu_sc as plsc

VOCAB = 100000
D = 64
BATCH = 4096
SEQ = 200

NC = 2
NS = 16
NW = NC * NS
TB = BATCH // NW

NBG = 6
NBB = 2


def _make_sc_gather():
    mesh = plsc.VectorSubcoreMesh(
        core_axis_name="c", subcore_axis_name="s", num_cores=NC, num_subcores=NS
    )

    @functools.partial(
        pl.kernel,
        mesh=mesh,
        out_type=jax.ShapeDtypeStruct((SEQ, D // 8, NW, 8, TB), jnp.float32),
        scratch_types=[
            pltpu.VMEM((SEQ, TB), jnp.int32),
            pltpu.VMEM((NBG, TB, D), jnp.float32),
            pltpu.VMEM((NBB, D // 8, 8, TB), jnp.float32),
        ]
        + [
            pltpu.SemaphoreType.DMA((NBG,)),
            pltpu.SemaphoreType.DMA((NBB,)),
        ],
        compiler_params=pltpu.CompilerParams(
            use_tc_tiling_on_sc=False, needs_layout_passes=False
        ),
    )
    def k(code_hbm, table_hbm, out_hbm, idx_v, rows_v, tile_v, gsem, wsem):
        wid = lax.axis_index("s") * NC + lax.axis_index("c")
        pltpu.sync_copy(code_hbm.at[wid], idx_v)

        def start_gather(s, g):
            pltpu.async_copy(
                table_hbm.at[idx_v.at[s]], rows_v.at[g], gsem.at[g]
            )

        def wait_gather(s, g):
            pltpu.make_async_copy(
                table_hbm.at[idx_v.at[s]], rows_v.at[g], gsem.at[g]
            ).wait()

        def start_write(s, b):
            pltpu.async_copy(tile_v.at[b], out_hbm.at[s, :, wid], wsem.at[b])

        def wait_write(s, b):
            pltpu.make_async_copy(
                tile_v.at[b], out_hbm.at[s, :, wid], wsem.at[b]
            ).wait()

        iota = lax.iota(jnp.int32, 16)

        def transpose(g, b):
            src = rows_v.at[g]
            dst = tile_v.at[b]

            @plsc.parallel_loop(0, TB // 16, 1, unroll=2)
            def _(c):
                rows = iota + c * 16
                for d0 in range(0, D, 8):
                    xs = [
                        plsc.load_gather(
                            src, [rows, jnp.full((16,), d0 + dd, jnp.int32)]
                        )
                        for dd in range(8)
                    ]
                    for dd in range(8):
                        d = d0 + dd
                        dst[d // 8, d % 8, pl.ds(c * 16, 16)] = xs[dd]

        for t in range(NBG - 1):
            start_gather(t, t)

        def stripe(j, carry):
            g = lax.rem(j, NBG)
            b = lax.rem(j, NBB)

            @pl.when(j + NBG - 1 < SEQ)
            def _():
                start_gather(j + NBG - 1, lax.rem(j + NBG - 1, NBG))

            wait_gather(j, g)

            @pl.when(j >= NBB)
            def _():
                wait_write(j - NBB, b)

            transpose(g, b)
            start_write(j, b)
            return carry

        lax.fori_loop(0, SEQ, stripe, 0)

        for j in range(SEQ - NBB, SEQ):
            wait_write(j, j % NBB)

    return k


_sc_gather = _make_sc_gather()


def kernel(code, embedding):
    code_t = code.reshape(NW, TB, SEQ).transpose(0, 2, 1).astype(jnp.int32)
    out5 = _sc_gather(code_t, embedding)
    return out5.transpose(2, 4, 0, 1, 3).reshape(BATCH, SEQ, D)


# speedup vs baseline: 1.0859x; 1.0859x over previous
import functools

import jax
import jax.numpy as jnp
from jax import lax
from jax.experimental import pallas as pl
from jax.experimental.pallas import tpu as pltpu
from jax.experimental.pallas import tpu_sc as plsc

VOCAB = 100000
D = 64
BATCH = 4096
SEQ = 200

NC = 2
NS = 16
NW = NC * NS

PER_W = BATCH // NW
NBUF = 6
GAHEAD = 3


def _make_sc_gather():
    mesh = plsc.VectorSubcoreMesh(
        core_axis_name="c", subcore_axis_name="s", num_cores=NC, num_subcores=NS
    )

    @functools.partial(
        pl.kernel,
        mesh=mesh,
        out_type=jax.ShapeDtypeStruct((BATCH, SEQ, D), jnp.float32),
        scratch_types=[
            pltpu.VMEM((PER_W, SEQ), jnp.int32),
            pltpu.VMEM((NBUF, SEQ, D), jnp.float32),
        ]
        + [pltpu.SemaphoreType.DMA] * NBUF
        + [pltpu.SemaphoreType.DMA] * NBUF,
        compiler_params=pltpu.CompilerParams(use_tc_tiling_on_sc=False),
    )
    def k(code_hbm, table_hbm, out_hbm, idx_v, rows_v, *sems):
        gsem = sems[:NBUF]
        wsem = sems[NBUF:]
        wid = lax.axis_index("s") * NC + lax.axis_index("c")
        bb = wid * PER_W
        pltpu.sync_copy(code_hbm.at[wid], idx_v)

        def start_gather(j, b):
            pltpu.async_copy(table_hbm.at[idx_v.at[j]], rows_v.at[b], gsem[b])

        def wait_gather(j, b):
            pltpu.make_async_copy(
                table_hbm.at[idx_v.at[j]], rows_v.at[b], gsem[b]
            ).wait()

        def start_write(j, b):
            pltpu.async_copy(rows_v.at[b], out_hbm.at[bb + j], wsem[b])

        def wait_write(j, b):
            pltpu.make_async_copy(
                rows_v.at[b], out_hbm.at[bb + j], wsem[b]
            ).wait()

        for t in range(GAHEAD):
            start_gather(t, t)
        for j in range(NBUF - GAHEAD):
            start_gather(j + GAHEAD, j + GAHEAD)
            wait_gather(j, j)
            start_write(j, j)

        J0 = NBUF - GAHEAD
        M = (PER_W - GAHEAD - J0) // NBUF

        def block(i, carry):
            for t in range(NBUF):
                j = J0 + i * NBUF + t
                b = (J0 + t) % NBUF
                bg = (J0 + t + GAHEAD) % NBUF
                wait_write(j + GAHEAD - NBUF, bg)
                start_gather(j + GAHEAD, bg)
                wait_gather(j, b)
                start_write(j, b)
            return carry

        lax.fori_loop(0, M, block, 0)

        for j in range(J0 + M * NBUF, PER_W):
            b = j % NBUF
            if j + GAHEAD < PER_W:
                bg = (j + GAHEAD) % NBUF
                wait_write(j + GAHEAD - NBUF, bg)
                start_gather(j + GAHEAD, bg)
            wait_gather(j, b)
            start_write(j, b)
        for j in range(PER_W - NBUF, PER_W):
            wait_write(j, j % NBUF)

    return k


_sc_gather = _make_sc_gather()


def kernel(code, embedding):
    code3 = code.reshape(NW, PER_W, SEQ).astype(jnp.int32)
    return _sc_gather(code3, embedding)
